# speedup vs baseline: 4.8170x; 4.8170x over previous
import numpy as np
import jax
import jax.numpy as jnp
from jax.experimental import pallas as pl
from jax.experimental.pallas import tpu as pltpu

_C3 = 1.0 / np.sqrt(3.0)
_C6 = 1.0 / np.sqrt(6.0)
_NNZ = [
    (1.0, 0, 0, 0, 0),
    (_C3, 0, 1, 1, 1), (_C3, 0, 2, 2, 1), (_C3, 0, 3, 3, 1),
    (_C3, 1, 0, 1, 2), (_C3, 1, 1, 0, 3), (_C6, 1, 2, 3, 4), (-_C6, 1, 3, 2, 4),
    (_C3, 2, 0, 2, 2), (_C3, 2, 2, 0, 3), (-_C6, 2, 1, 3, 4), (_C6, 2, 3, 1, 4),
    (_C3, 3, 0, 3, 2), (_C3, 3, 3, 0, 3), (_C6, 3, 1, 2, 4), (-_C6, 3, 2, 1, 4),
]

_T_NP = np.zeros((4, 4, 4, 5), dtype=np.float32)
for _v, _mo, _mi, _me, _w in _NNZ:
    _T_NP[_me, _mi, _mo, _w] = _v

_BB = 2000


def _so3_kernel(e_ref, f_ref, w_ref, o_ref):
    f = f_ref[...]
    e = e_ref[...]
    p = jnp.concatenate([e[:, i:i + 1] * f for i in range(4)], axis=1)
    o_ref[...] = jnp.dot(p, w_ref[...], preferred_element_type=jnp.float32)


def kernel(feature, edge_feat, weight):
    B = feature.shape[0]
    f_flat = feature.reshape(B, 4 * 64)
    t = jnp.asarray(_T_NP)
    wbig = jnp.einsum('eiow,wcd->eicod', t, weight[0]).reshape(1024, 256)

    grid = B // _BB
    out = pl.pallas_call(
        _so3_kernel,
        grid=(grid,),
        in_specs=[
            pl.BlockSpec((_BB, 4), lambda i: (i, 0)),
            pl.BlockSpec((_BB, 256), lambda i: (i, 0)),
            pl.BlockSpec((1024, 256), lambda i: (0, 0)),
        ],
        out_specs=pl.BlockSpec((_BB, 256), lambda i: (i, 0)),
        out_shape=jax.ShapeDtypeStruct((B, 256), jnp.float32),
        compiler_params=pltpu.CompilerParams(
            dimension_semantics=("arbitrary",),
        ),
    )(edge_feat, f_flat, wbig)
    return out.reshape(B, 4, 64)


# speedup vs baseline: 16.8780x; 3.5038x over previous
import numpy as np
import jax
import jax.numpy as jnp
from jax.experimental import pallas as pl
from jax.experimental.pallas import tpu as pltpu

_C3 = 1.0 / np.sqrt(3.0)
_C6 = 1.0 / np.sqrt(6.0)
_NNZ = [
    (1.0, 0, 0, 0, 0),
    (_C3, 0, 1, 1, 1), (_C3, 0, 2, 2, 1), (_C3, 0, 3, 3, 1),
    (_C3, 1, 0, 1, 2), (_C3, 1, 1, 0, 3), (_C6, 1, 2, 3, 4), (-_C6, 1, 3, 2, 4),
    (_C3, 2, 0, 2, 2), (_C3, 2, 2, 0, 3), (-_C6, 2, 1, 3, 4), (_C6, 2, 3, 1, 4),
    (_C3, 3, 0, 3, 2), (_C3, 3, 3, 0, 3), (_C6, 3, 1, 2, 4), (-_C6, 3, 2, 1, 4),
]

_T_NP = np.zeros((4, 4, 4, 5), dtype=np.float32)
for _v, _mo, _mi, _me, _w in _NNZ:
    _T_NP[_me, _mi, _mo, _w] = _v

_BB = 12544


def _so3_kernel(e_ref, f_ref, w_ref, o_ref):
    bb = f_ref.shape[2]
    f = f_ref[...].reshape(256, bb).astype(jnp.bfloat16)
    e = e_ref[...].astype(jnp.bfloat16)
    p = jnp.concatenate(
        [jnp.broadcast_to(e[i:i + 1, :], (256, bb)) * f for i in range(4)],
        axis=0)
    o = jnp.dot(w_ref[...], p, preferred_element_type=jnp.float32)
    o_ref[...] = o.reshape(4, 64, bb)


def kernel(feature, edge_feat, weight):
    B = feature.shape[0]
    t5 = jnp.asarray(_T_NP.transpose(2, 0, 1, 3))
    wtr = weight[0].transpose(1, 2, 0)
    wt = ((t5[:, None, :, :, None, :] * wtr.transpose(1, 0, 2)[None, :, None, None, :, :])
          .sum(-1).reshape(256, 1024).astype(jnp.bfloat16))
    ft = feature.transpose(1, 2, 0)
    et = edge_feat.transpose(1, 0)

    grid = (B + _BB - 1) // _BB
    out_t = pl.pallas_call(
        _so3_kernel,
        grid=(grid,),
        in_specs=[
            pl.BlockSpec((4, _BB), lambda i: (0, i)),
            pl.BlockSpec((4, 64, _BB), lambda i: (0, 0, i)),
            pl.BlockSpec((256, 1024), lambda i: (0, 0)),
        ],
        out_specs=pl.BlockSpec((4, 64, _BB), lambda i: (0, 0, i)),
        out_shape=jax.ShapeDtypeStruct((4, 64, B), jnp.float32),
        compiler_params=pltpu.CompilerParams(
            dimension_semantics=("parallel",),
        ),
    )(et, ft, wt)
    return out_t.transpose(2, 0, 1)


# speedup vs baseline: 17.1579x; 1.0166x over previous
import numpy as np
import jax
import jax.numpy as jnp
from jax.experimental import pallas as pl
from jax.experimental.pallas import tpu as pltpu

_C3 = 1.0 / np.sqrt(3.0)
_C6 = 1.0 / np.sqrt(6.0)
_NNZ = [
    (1.0, 0, 0, 0, 0),
    (_C3, 0, 1, 1, 1), (_C3, 0, 2, 2, 1), (_C3, 0, 3, 3, 1),
    (_C3, 1, 0, 1, 2), (_C3, 1, 1, 0, 3), (_C6, 1, 2, 3, 4), (-_C6, 1, 3, 2, 4),
    (_C3, 2, 0, 2, 2), (_C3, 2, 2, 0, 3), (-_C6, 2, 1, 3, 4), (_C6, 2, 3, 1, 4),
    (_C3, 3, 0, 3, 2), (_C3, 3, 3, 0, 3), (_C6, 3, 1, 2, 4), (-_C6, 3, 2, 1, 4),
]

_T_NP = np.zeros((4, 4, 4, 5), dtype=np.float32)
for _v, _mo, _mi, _me, _w in _NNZ:
    _T_NP[_me, _mi, _mo, _w] = _v

_BB = 6272


def _so3_kernel(e_ref, f_ref, w_ref, o_ref):
    bb = f_ref.shape[2]
    f = f_ref[...].reshape(256, bb).astype(jnp.bfloat16)
    e = e_ref[...].astype(jnp.bfloat16)
    p = jnp.concatenate(
        [jnp.broadcast_to(e[i:i + 1, :], (256, bb)) * f for i in range(4)],
        axis=0)
    o = jnp.dot(w_ref[...], p, preferred_element_type=jnp.float32)
    o_ref[...] = o.reshape(4, 64, bb)


def kernel(feature, edge_feat, weight):
    B = feature.shape[0]
    t5 = jnp.asarray(_T_NP.transpose(2, 0, 1, 3))
    wtr = weight[0].transpose(1, 2, 0)
    wt = ((t5[:, None, :, :, None, :] * wtr.transpose(1, 0, 2)[None, :, None, None, :, :])
          .sum(-1).reshape(256, 1024).astype(jnp.bfloat16))
    ft = feature.transpose(1, 2, 0)
    et = edge_feat.transpose(1, 0)

    grid = (B + _BB - 1) // _BB
    out_t = pl.pallas_call(
        _so3_kernel,
        grid=(grid,),
        in_specs=[
            pl.BlockSpec((4, _BB), lambda i: (0, i)),
            pl.BlockSpec((4, 64, _BB), lambda i: (0, 0, i)),
            pl.BlockSpec((256, 1024), lambda i: (0, 0)),
        ],
        out_specs=pl.BlockSpec((4, 64, _BB), lambda i: (0, 0, i)),
        out_shape=jax.ShapeDtypeStruct((4, 64, B), jnp.float32),
        compiler_params=pltpu.CompilerParams(
            dimension_semantics=("parallel",),
        ),
    )(et, ft, wt)
    return out_t.transpose(2, 0, 1)
